# baseline (speedup 1.0000x reference)
import jax
import jax.numpy as jnp
from jax import lax
from jax.experimental import pallas as pl
from jax.experimental.pallas import tpu as pltpu
from jax.experimental.pallas import tpu_sc as plsc

VOCAB = 100000
D_MODEL = 1024
NUM_TOKENS = 4 * 4096

_NC = 2
_NS = 16
_NW = _NC * _NS
_B_PER_W = NUM_TOKENS // _NW
_CHUNK = 16
_NCHUNK = _B_PER_W // _CHUNK
_NBUF = 6


def _gather_body(idx_hbm, table_hbm, out_hbm, idx_v, *rest):
    bufs = rest[:_NBUF]
    gsems = rest[_NBUF:2 * _NBUF]
    ssems = rest[2 * _NBUF:3 * _NBUF]
    wid = lax.axis_index("s") * _NC + lax.axis_index("c")
    base = wid * _B_PER_W
    pltpu.sync_copy(idx_hbm.at[pl.ds(base, _B_PER_W)], idx_v)

    def start_gather(j):
        pltpu.async_copy(
            table_hbm.at[idx_v.at[pl.ds(j * _CHUNK, _CHUNK)]],
            bufs[j % _NBUF],
            gsems[j % _NBUF],
        )

    def gather_done(j):
        pltpu.make_async_copy(
            table_hbm.at[idx_v.at[pl.ds(j * _CHUNK, _CHUNK)]],
            bufs[j % _NBUF],
            gsems[j % _NBUF],
        ).wait()

    def start_store(j):
        pltpu.async_copy(
            bufs[j % _NBUF],
            out_hbm.at[pl.ds(base + j * _CHUNK, _CHUNK)],
            ssems[j % _NBUF],
        )

    def store_done(j):
        pltpu.make_async_copy(
            bufs[j % _NBUF],
            out_hbm.at[pl.ds(base + j * _CHUNK, _CHUNK)],
            ssems[j % _NBUF],
        ).wait()

    for k in range(_NBUF - 1):
        start_gather(k)
    for j in range(_NCHUNK):
        gather_done(j)
        start_store(j)
        k = j + _NBUF - 1
        if k < _NCHUNK:
            if j >= 1:
                store_done(j - 1)
            start_gather(k)
    for j in range(max(0, _NCHUNK - _NBUF), _NCHUNK):
        store_done(j)


_gather = pl.kernel(
    _gather_body,
    out_type=jax.ShapeDtypeStruct((NUM_TOKENS, D_MODEL), jnp.float32),
    mesh=plsc.VectorSubcoreMesh(core_axis_name="c", subcore_axis_name="s"),
    scratch_types=(
        [pltpu.VMEM((_B_PER_W,), jnp.int32)]
        + [pltpu.VMEM((_CHUNK, D_MODEL), jnp.float32) for _ in range(_NBUF)]
        + [pltpu.SemaphoreType.DMA for _ in range(2 * _NBUF)]
    ),
)


@jax.jit
def kernel(token_ids, weights):
    flat_ids = token_ids.reshape(-1).astype(jnp.int32)
    out = _gather(flat_ids, weights)
    return out.reshape(*token_ids.shape, D_MODEL)
